# Initial kernel scaffold; baseline (speedup 1.0000x reference)
import jax
import jax.numpy as jnp
from jax.experimental import pallas as pl


def kernel(traversal_lists, adj_matrices, ent_attn, spo_attn, ctx_idx_adjusted, roi_cls, roi_mask, weight_on_children):
    raise NotImplementedError("write your pallas kernel here")


# speedup vs baseline: 462.9684x; 462.9684x over previous
import functools

import jax
import jax.numpy as jnp
from jax import lax
from jax.experimental import pallas as pl
from jax.experimental.pallas import tpu as pltpu
from jax.experimental.pallas import tpu_sc as plsc

NC, NS, L = 2, 16, 16
NW = NC * NS

BS, NSEQ, SSEQ, NBOX, NCTX = 64, 12, 12, 196, 16
MAXLEN = 12
JPAD = 208
NJC = JPAD // L
EPW = JPAD * NCTX
BPW = SSEQ * EPW


def _w_body(spo_ref, mm_ref, km_ref, o_ref):
    m = mm_ref[...]
    o_ref[...] = spo_ref[...] * (m * m * m) * km_ref[...]


def _fuse_w(spo_flat, mm_flat, km_rep):
    return pl.pallas_call(
        _w_body,
        out_shape=jax.ShapeDtypeStruct((BS, SSEQ, EPW), jnp.float32),
        grid=(BS,),
        in_specs=[
            pl.BlockSpec((1, SSEQ, EPW), lambda b: (b, 0, 0)),
            pl.BlockSpec((1, SSEQ, EPW), lambda b: (b, 0, 0)),
            pl.BlockSpec((1, 1, EPW), lambda b: (b, 0, 0)),
        ],
        out_specs=pl.BlockSpec((1, SSEQ, EPW), lambda b: (b, 0, 0)),
    )(spo_flat, mm_flat, km_rep)


def _sread(ref, idx):
    return ref[pl.ds(idx, L)][0]


def _sc_body(ctx_hbm, w_hbm, ent_hbm, wp_hbm, km_hbm, adj_hbm, par_hbm,
             out_hbm, ctxb, wb, Eb, Mb, kmb, wpb, adjb, parb, Ub):
    wid = lax.axis_index("s") * NC + lax.axis_index("c")
    striota = lax.iota(jnp.int32, L) * NCTX

    def run_batch(b):
        pltpu.sync_copy(ctx_hbm.at[b], ctxb)
        pltpu.sync_copy(w_hbm.at[b], wb)
        pltpu.sync_copy(ent_hbm.at[b], Eb)
        pltpu.sync_copy(wp_hbm.at[b], wpb)
        pltpu.sync_copy(km_hbm.at[b], kmb)
        pltpu.sync_copy(adj_hbm.at[b], adjb)
        pltpu.sync_copy(par_hbm.at[b], parb)

        for c in range(NSEQ):
            for jc in range(NJC):
                sl = pl.ds(jc * L, L)
                Mb[c, sl] = Eb[c, sl] * kmb[sl]

        def iter_body(it, carry):
            p = _sread(parb, it)

            def c_body(c, accs):
                e = _sread(adjb, p * NSEQ + c)
                crow = jnp.full((L,), c, jnp.int32)
                base = striota + e * EPW
                new = []
                for jc in range(NJC):
                    acc = accs[jc]
                    bvec = base + jc * (L * NCTX)
                    for t in range(NCTX):
                        ivec = bvec + t
                        idx = plsc.load_gather(ctxb, [ivec])
                        wv = plsc.load_gather(wb, [ivec])
                        g = plsc.load_gather(Mb, [crow, idx])
                        acc = acc + g * wv
                    new.append(acc)
                return tuple(new)

            zero = jnp.zeros((L,), jnp.float32)
            accs = lax.fori_loop(0, NSEQ, c_body, (zero,) * NJC)

            mx = jnp.zeros((L,), jnp.float32)
            bias = jnp.float32(NSEQ) * jnp.float32(1e-6)
            for jc in range(NJC):
                sl = pl.ds(jc * L, L)
                u = Eb[p, sl] + (accs[jc] + bias) * wpb[p, sl]
                Ub[sl] = u
                mx = jnp.maximum(mx, jnp.abs(u))
            norm = jnp.max(mx)
            norm = jnp.maximum(norm, jnp.float32(1.0))
            invv = jnp.full((L,), jnp.float32(1.0)) / jnp.full((L,), norm)
            for jc in range(NJC):
                sl = pl.ds(jc * L, L)
                u = Ub[sl] * invv
                Eb[p, sl] = u
                Mb[p, sl] = u * kmb[sl]
            return carry

        lax.fori_loop(0, MAXLEN, iter_body, 0)
        pltpu.sync_copy(Eb, out_hbm.at[b])

    run_batch(wid)
    run_batch(wid + NW)


def kernel(traversal_lists, adj_matrices, ent_attn, spo_attn,
           ctx_idx_adjusted, roi_cls, roi_mask, weight_on_children):
    f32 = jnp.float32
    pj = JPAD - NBOX

    km = (roi_cls != -1.0).astype(f32)
    km_pad = jnp.pad(km, ((0, 0), (0, pj)))
    km_rep = jnp.repeat(km_pad, NCTX, axis=-1).reshape(BS, 1, EPW)

    spo_flat = jnp.pad(spo_attn, ((0, 0), (0, 0), (0, pj), (0, 0))
                       ).reshape(BS, SSEQ, EPW)
    mm_flat = jnp.pad(roi_mask, ((0, 0), (0, 0), (0, pj), (0, 0))
                      ).reshape(BS, SSEQ, EPW)
    ctx_flat = jnp.pad(ctx_idx_adjusted, ((0, 0), (0, 0), (0, pj), (0, 0))
                       ).reshape(BS, BPW)

    w_flat = _fuse_w(spo_flat, mm_flat, km_rep).reshape(BS, BPW)

    ent_pad = jnp.pad(ent_attn, ((0, 0), (0, 0), (0, pj)))
    wp_pad = jnp.pad(weight_on_children, ((0, 0), (0, 0), (0, pj)))
    adj_flat = jnp.pad(adj_matrices.reshape(BS, NSEQ * NSEQ),
                       ((0, 0), (0, 160 - NSEQ * NSEQ)))
    par_pad = jnp.pad(traversal_lists, ((0, 0), (0, 20)))

    mesh = plsc.VectorSubcoreMesh(core_axis_name="c", subcore_axis_name="s",
                                  num_cores=NC, num_subcores=NS)
    sc = functools.partial(
        pl.kernel,
        out_type=jax.ShapeDtypeStruct((BS, NSEQ, JPAD), jnp.float32),
        mesh=mesh,
        compiler_params=pltpu.CompilerParams(needs_layout_passes=False),
        scratch_types=[
            pltpu.VMEM((BPW,), jnp.int32),
            pltpu.VMEM((BPW,), jnp.float32),
            pltpu.VMEM((NSEQ, JPAD), jnp.float32),
            pltpu.VMEM((NSEQ, JPAD), jnp.float32),
            pltpu.VMEM((JPAD,), jnp.float32),
            pltpu.VMEM((NSEQ, JPAD), jnp.float32),
            pltpu.VMEM((160,), jnp.int32),
            pltpu.VMEM((32,), jnp.int32),
            pltpu.VMEM((JPAD,), jnp.float32),
        ],
    )(_sc_body)

    out = sc(ctx_flat, w_flat, ent_pad, wp_pad, km_pad, adj_flat, par_pad)
    return out[:, :, :NBOX]


# speedup vs baseline: 691.5539x; 1.4937x over previous
import functools

import jax
import jax.numpy as jnp
from jax import lax
from jax.experimental import pallas as pl
from jax.experimental.pallas import tpu as pltpu
from jax.experimental.pallas import tpu_sc as plsc

NC, NS, L = 2, 16, 16
NW = NC * NS

BS, NSEQ, SSEQ, NBOX, NCTX = 64, 12, 12, 196, 16
MAXLEN = 12
JPAD = 208
NJC = JPAD // L
EPW = JPAD * NCTX
BPW = SSEQ * EPW


def _pack_body(spo_ref, mm_ref, ctx_ref, km_ref, o_ref):
    m = mm_ref[...]
    w = spo_ref[...] * (m * m * m) * km_ref[...][:, :, :NBOX, None]
    b = lax.bitcast_convert_type(w, jnp.int32)
    r = lax.shift_right_logical(b, 16) & 1
    b16 = lax.shift_right_logical(b + 0x7FFF + r, 16)
    packed = jnp.left_shift(ctx_ref[...], 16) | b16
    packed_t = jnp.swapaxes(packed, -1, -2)
    o_ref[:, :, :, :NBOX] = packed_t
    o_ref[:, :, :, NBOX:] = jnp.zeros((1, SSEQ, NCTX, JPAD - NBOX),
                                      jnp.int32)


def _pack_tables(spo, mm, ctx, km_pad):
    blk_in = pl.BlockSpec((1, SSEQ, NBOX, NCTX), lambda b: (b, 0, 0, 0))
    return pl.pallas_call(
        _pack_body,
        out_shape=jax.ShapeDtypeStruct((BS, SSEQ, NCTX, JPAD), jnp.int32),
        grid=(BS,),
        in_specs=[blk_in, blk_in, blk_in,
                  pl.BlockSpec((1, 1, JPAD), lambda b: (b, 0, 0))],
        out_specs=pl.BlockSpec((1, SSEQ, NCTX, JPAD), lambda b: (b, 0, 0, 0)),
    )(spo, mm, ctx, km_pad)


def _sread(ref, idx):
    return ref[pl.ds(idx, L)][0]


def _sc_body(pw_hbm, ent_hbm, wp_hbm, km_hbm, adj_hbm, par_hbm,
             out_hbm, pwb, Eb, Mb, kmb, wpb, adjb, parb, Ub):
    wid = lax.axis_index("s") * NC + lax.axis_index("c")

    def run_batch(b):
        pltpu.sync_copy(pw_hbm.at[b], pwb)
        pltpu.sync_copy(ent_hbm.at[b], Eb)
        pltpu.sync_copy(wp_hbm.at[b], wpb)
        pltpu.sync_copy(km_hbm.at[b], kmb)
        pltpu.sync_copy(adj_hbm.at[b], adjb)
        pltpu.sync_copy(par_hbm.at[b], parb)

        for c in range(NSEQ):
            for jc in range(NJC):
                sl = pl.ds(jc * L, L)
                Mb[c, sl] = Eb[c, sl] * kmb[sl]

        def iter_body(it, carry):
            p = _sread(parb, it)

            def c_body(c, accs):
                e = _sread(adjb, p * NSEQ + c)
                crow = jnp.full((L,), c, jnp.int32)
                new = []
                for jc in range(NJC):
                    acc = accs[jc]
                    sl = pl.ds(jc * L, L)
                    for t in range(NCTX):
                        wvec = pwb[e, t, sl]
                        idx = lax.shift_right_logical(wvec, 16)
                        wf = plsc.bitcast(jnp.left_shift(wvec, 16),
                                          jnp.float32)
                        g = plsc.load_gather(Mb, [crow, idx])
                        acc = acc + g * wf
                    new.append(acc)
                return tuple(new)

            zero = jnp.zeros((L,), jnp.float32)
            accs = lax.fori_loop(0, NSEQ, c_body, (zero,) * NJC)

            mx = jnp.zeros((L,), jnp.float32)
            bias = jnp.float32(NSEQ) * jnp.float32(1e-6)
            for jc in range(NJC):
                sl = pl.ds(jc * L, L)
                u = Eb[p, sl] + (accs[jc] + bias) * wpb[p, sl]
                Ub[sl] = u
                mx = jnp.maximum(mx, jnp.abs(u))
            norm = jnp.max(mx)
            norm = jnp.maximum(norm, jnp.float32(1.0))
            invv = jnp.full((L,), jnp.float32(1.0)) / jnp.full((L,), norm)
            for jc in range(NJC):
                sl = pl.ds(jc * L, L)
                u = Ub[sl] * invv
                Eb[p, sl] = u
                Mb[p, sl] = u * kmb[sl]
            return carry

        lax.fori_loop(0, MAXLEN, iter_body, 0)
        pltpu.sync_copy(Eb, out_hbm.at[b])

    run_batch(wid)
    run_batch(wid + NW)


def kernel(traversal_lists, adj_matrices, ent_attn, spo_attn,
           ctx_idx_adjusted, roi_cls, roi_mask, weight_on_children):
    f32 = jnp.float32
    pj = JPAD - NBOX

    km = (roi_cls != -1.0).astype(f32)
    km_pad = jnp.pad(km, ((0, 0), (0, pj)))

    packed = _pack_tables(spo_attn, roi_mask, ctx_idx_adjusted,
                          km_pad.reshape(BS, 1, JPAD))

    ent_pad = jnp.pad(ent_attn, ((0, 0), (0, 0), (0, pj)))
    wp_pad = jnp.pad(weight_on_children, ((0, 0), (0, 0), (0, pj)))
    adj_flat = jnp.pad(adj_matrices.reshape(BS, NSEQ * NSEQ),
                       ((0, 0), (0, 160 - NSEQ * NSEQ)))
    par_pad = jnp.pad(traversal_lists, ((0, 0), (0, 20)))

    mesh = plsc.VectorSubcoreMesh(core_axis_name="c", subcore_axis_name="s",
                                  num_cores=NC, num_subcores=NS)
    sc = functools.partial(
        pl.kernel,
        out_type=jax.ShapeDtypeStruct((BS, NSEQ, JPAD), jnp.float32),
        mesh=mesh,
        compiler_params=pltpu.CompilerParams(needs_layout_passes=False),
        scratch_types=[
            pltpu.VMEM((SSEQ, NCTX, JPAD), jnp.int32),
            pltpu.VMEM((NSEQ, JPAD), jnp.float32),
            pltpu.VMEM((NSEQ, JPAD), jnp.float32),
            pltpu.VMEM((JPAD,), jnp.float32),
            pltpu.VMEM((NSEQ, JPAD), jnp.float32),
            pltpu.VMEM((160,), jnp.int32),
            pltpu.VMEM((32,), jnp.int32),
            pltpu.VMEM((JPAD,), jnp.float32),
        ],
    )(_sc_body)

    out = sc(packed, ent_pad, wp_pad, km_pad, adj_flat, par_pad)
    return out[:, :, :NBOX]


# speedup vs baseline: 1186.1583x; 1.7152x over previous
import functools

import jax
import jax.numpy as jnp
from jax import lax
from jax.experimental import pallas as pl
from jax.experimental.pallas import tpu as pltpu
from jax.experimental.pallas import tpu_sc as plsc

NC, NS, L = 2, 16, 16
NW = NC * NS

BS, NSEQ, SSEQ, NBOX, NCTX = 64, 12, 12, 196, 16
MAXLEN = 12
JPAD = 208
NJC = JPAD // L
EPW = JPAD * NCTX
BPW = SSEQ * EPW


def _pack_body(spo_ref, mm_ref, ctx_ref, km_ref, o_ref):
    m = mm_ref[...]
    w = spo_ref[...] * (m * m * m) * km_ref[...][:, :, None, :NBOX]
    b = lax.bitcast_convert_type(w, jnp.int32)
    r = lax.shift_right_logical(b, 16) & 1
    b16 = lax.shift_right_logical(b + 0x7FFF + r, 16)
    packed = jnp.left_shift(ctx_ref[...], 16) | b16
    o_ref[:, :, :, :NBOX] = packed
    o_ref[:, :, :, NBOX:] = jnp.zeros((1, SSEQ, NCTX, JPAD - NBOX),
                                      jnp.int32)


def _pack_tables(spo_t, mm_t, ctx_t, km_pad):
    blk_in = pl.BlockSpec((1, SSEQ, NCTX, NBOX), lambda b: (b, 0, 0, 0))
    return pl.pallas_call(
        _pack_body,
        out_shape=jax.ShapeDtypeStruct((BS, SSEQ, NCTX, JPAD), jnp.int32),
        grid=(BS,),
        in_specs=[blk_in, blk_in, blk_in,
                  pl.BlockSpec((1, 1, JPAD), lambda b: (b, 0, 0))],
        out_specs=pl.BlockSpec((1, SSEQ, NCTX, JPAD), lambda b: (b, 0, 0, 0)),
    )(spo_t, mm_t, ctx_t, km_pad)


def _sread(ref, idx):
    return ref[pl.ds(idx, L)][0]


def _sc_body(pw_hbm, ent_hbm, wp_hbm, km_hbm, adj_hbm, par_hbm,
             out_hbm, pwb, Eb, Mb, kmb, wpb, adjb, parb, Ub):
    wid = lax.axis_index("s") * NC + lax.axis_index("c")

    def run_batch(b):
        pltpu.sync_copy(pw_hbm.at[b], pwb)
        pltpu.sync_copy(ent_hbm.at[b], Eb)
        pltpu.sync_copy(wp_hbm.at[b], wpb)
        pltpu.sync_copy(km_hbm.at[b], kmb)
        pltpu.sync_copy(adj_hbm.at[b], adjb)
        pltpu.sync_copy(par_hbm.at[b], parb)

        for c in range(NSEQ):
            for jc in range(NJC):
                sl = pl.ds(jc * L, L)
                Mb[c, sl] = Eb[c, sl] * kmb[sl]

        def iter_body(it, carry):
            p = _sread(parb, it)

            def c_body(c, accs):
                e = _sread(adjb, p * NSEQ + c)
                crow = jnp.full((L,), c, jnp.int32)
                new = []
                for jc in range(NJC):
                    a0, a1 = accs[jc]
                    sl = pl.ds(jc * L, L)
                    for t in range(NCTX):
                        wvec = pwb[e, t, sl]
                        idx = lax.shift_right_logical(wvec, 16)
                        wf = plsc.bitcast(jnp.left_shift(wvec, 16),
                                          jnp.float32)
                        g = plsc.load_gather(Mb, [crow, idx])
                        if t % 2 == 0:
                            a0 = a0 + g * wf
                        else:
                            a1 = a1 + g * wf
                    new.append((a0, a1))
                return tuple(new)

            zero = jnp.zeros((L,), jnp.float32)
            accs2 = lax.fori_loop(0, NSEQ, c_body, ((zero, zero),) * NJC)
            accs = [a0 + a1 for (a0, a1) in accs2]

            mx = jnp.zeros((L,), jnp.float32)
            bias = jnp.float32(NSEQ) * jnp.float32(1e-6)
            for jc in range(NJC):
                sl = pl.ds(jc * L, L)
                u = Eb[p, sl] + (accs[jc] + bias) * wpb[p, sl]
                Ub[sl] = u
                mx = jnp.maximum(mx, jnp.abs(u))
            norm = jnp.max(mx)
            norm = jnp.maximum(norm, jnp.float32(1.0))
            invv = jnp.full((L,), jnp.float32(1.0)) / jnp.full((L,), norm)
            for jc in range(NJC):
                sl = pl.ds(jc * L, L)
                u = Ub[sl] * invv
                Eb[p, sl] = u
                Mb[p, sl] = u * kmb[sl]
            return carry

        lax.fori_loop(0, MAXLEN, iter_body, 0)
        pltpu.sync_copy(Eb, out_hbm.at[b])

    run_batch(wid)
    run_batch(wid + NW)


def kernel(traversal_lists, adj_matrices, ent_attn, spo_attn,
           ctx_idx_adjusted, roi_cls, roi_mask, weight_on_children):
    f32 = jnp.float32
    pj = JPAD - NBOX

    km = (roi_cls != -1.0).astype(f32)
    km_pad = jnp.pad(km, ((0, 0), (0, pj)))

    packed = _pack_tables(jnp.swapaxes(spo_attn, 2, 3),
                          jnp.swapaxes(roi_mask, 2, 3),
                          jnp.swapaxes(ctx_idx_adjusted, 2, 3),
                          km_pad.reshape(BS, 1, JPAD))

    ent_pad = jnp.pad(ent_attn, ((0, 0), (0, 0), (0, pj)))
    wp_pad = jnp.pad(weight_on_children, ((0, 0), (0, 0), (0, pj)))
    adj_flat = jnp.pad(adj_matrices.reshape(BS, NSEQ * NSEQ),
                       ((0, 0), (0, 160 - NSEQ * NSEQ)))
    par_pad = jnp.pad(traversal_lists, ((0, 0), (0, 20)))

    mesh = plsc.VectorSubcoreMesh(core_axis_name="c", subcore_axis_name="s",
                                  num_cores=NC, num_subcores=NS)
    sc = functools.partial(
        pl.kernel,
        out_type=jax.ShapeDtypeStruct((BS, NSEQ, JPAD), jnp.float32),
        mesh=mesh,
        compiler_params=pltpu.CompilerParams(needs_layout_passes=False),
        scratch_types=[
            pltpu.VMEM((SSEQ, NCTX, JPAD), jnp.int32),
            pltpu.VMEM((NSEQ, JPAD), jnp.float32),
            pltpu.VMEM((NSEQ, JPAD), jnp.float32),
            pltpu.VMEM((JPAD,), jnp.float32),
            pltpu.VMEM((NSEQ, JPAD), jnp.float32),
            pltpu.VMEM((160,), jnp.int32),
            pltpu.VMEM((32,), jnp.int32),
            pltpu.VMEM((JPAD,), jnp.float32),
        ],
    )(_sc_body)

    out = sc(packed, ent_pad, wp_pad, km_pad, adj_flat, par_pad)
    return out[:, :, :NBOX]
